# baseline (speedup 1.0000x reference)
import functools

import jax
import jax.numpy as jnp
from jax import lax
from jax.experimental import pallas as pl
from jax.experimental.pallas import tpu as pltpu
from jax.experimental.pallas import tpu_sc as plsc

N = 10000
E = 320000
D = 128
R = 200
RPAD = 256
EPS = 1e-5

NC, NS, L = 2, 16, 16
NW = NC * NS
CH = 128
SLAB = 8
EPT = 10240
EP = NW * EPT
NSLAB = EPT // (SLAB * CH)
N2 = 10016
RQ = 624
RTAIL = N2 - RQ * NS
ZR = 8
CH2 = CH // 2
CH4 = CH // 4


def _tanh(v):
    return 1.0 - 2.0 / (jnp.exp(2.0 * v) + 1.0)


def _prologue_body(x_ref, w1t_ref, w2t_ref, am_ref, aq_ref, que_ref, bm_ref,
                   r_ref, wrt_ref, brl_ref, rg_ref, rb_ref,
                   xm_ref, ax_ref, rm_ref, arc_ref, rnew_ref):
    i = pl.program_id(0)
    xb = x_ref[...]
    xm = jnp.dot(xb, w1t_ref[...], preferred_element_type=jnp.float32)
    xm_ref[...] = xm
    ax_ref[...] = jnp.dot(xm, am_ref[...], preferred_element_type=jnp.float32)

    @pl.when(i == 0)
    def _():
        rb = r_ref[...]
        rm = jnp.dot(rb, w2t_ref[...], preferred_element_type=jnp.float32) \
            + bm_ref[...]
        rm_ref[...] = rm
        c0 = jnp.sum(que_ref[...] * aq_ref[...])
        arc_ref[...] = jnp.dot(rm, am_ref[...],
                               preferred_element_type=jnp.float32) + c0
        rl = jnp.dot(rb, wrt_ref[...], preferred_element_type=jnp.float32) \
            + brl_ref[...]
        mu = jnp.mean(rl, axis=0, keepdims=True)
        var = jnp.mean((rl - mu) ** 2, axis=0, keepdims=True)
        rnew_ref[...] = jnp.tanh(
            (rl - mu) / jnp.sqrt(var + EPS) * rg_ref[...] + rb_ref[...])


def _prologue(x, w1t, w2t, am, aq, que, bm, r, wrt, brl, rg, rb):
    grid = (N // 1000,)
    full = lambda shp: pl.BlockSpec(shp, lambda i: (0, 0))
    return pl.pallas_call(
        _prologue_body,
        grid=grid,
        in_specs=[
            pl.BlockSpec((1000, D), lambda i: (i, 0)),
            full((D, D)), full((D, D)), full((D, 1)), full((1, D)),
            full((1, D)), full((1, D)),
            full((R, D)), full((D, D)), full((1, D)), full((1, D)),
            full((1, D)),
        ],
        out_specs=[
            pl.BlockSpec((1000, D), lambda i: (i, 0)),
            pl.BlockSpec((1000, 1), lambda i: (i, 0)),
            full((R, D)), full((R, 1)), full((R, D)),
        ],
        out_shape=[
            jax.ShapeDtypeStruct((N, D), jnp.float32),
            jax.ShapeDtypeStruct((N, 1), jnp.float32),
            jax.ShapeDtypeStruct((R, D), jnp.float32),
            jax.ShapeDtypeStruct((R, 1), jnp.float32),
            jax.ShapeDtypeStruct((R, D), jnp.float32),
        ],
    )(x, w1t, w2t, am, aq, que, bm, r, wrt, brl, rg, rb)


def _sc_mesh():
    return plsc.VectorSubcoreMesh(core_axis_name="c", subcore_axis_name="s",
                                  num_cores=NC, num_subcores=NS)


def _atten_body(e3_hbm, ax_hbm, arc_hbm, zn_hbm,
                atten_out, coeffs_out,
                comb_v, atten_v, ax_v, arc_v, coeffs_sh, sem):
    c = lax.axis_index("c")
    s = lax.axis_index("s")
    wid = c * NS + s
    pltpu.sync_copy(ax_hbm, ax_v)
    pltpu.sync_copy(arc_hbm, arc_v)

    @pl.when(s == 0)
    def _():
        pltpu.sync_copy(zn_hbm, coeffs_sh)
    plsc.subcore_barrier()

    def slab_body(j, _):
        sl8 = pl.ds(j * SLAB, SLAB)
        pltpu.sync_copy(e3_hbm.at[wid, j], comb_v)

        def row_body(i, _):
            def vec_body(g, _):
                sl = pl.ds(g * L, L)
                axg = plsc.load_gather(ax_v, [comb_v[i, sl]])
                arg = plsc.load_gather(arc_v, [comb_v[SLAB + i, sl]])
                atten_v[i, sl] = jnp.exp(_tanh(axg + arg))
                return 0
            lax.fori_loop(0, CH // L, vec_body, 0)
            pltpu.async_copy(atten_v.at[i],
                             coeffs_sh.at[comb_v.at[2 * SLAB + i]],
                             sem, add=True)
            return 0
        lax.fori_loop(0, SLAB, row_body, 0)

        def drain_body(i, _):
            pltpu.make_async_copy(
                atten_v.at[i], coeffs_sh.at[comb_v.at[2 * SLAB + i]],
                sem).wait()
            return 0
        lax.fori_loop(0, SLAB, drain_body, 0)
        pltpu.sync_copy(atten_v, atten_out.at[wid, sl8])
        return 0
    lax.fori_loop(0, NSLAB, slab_body, 0)

    plsc.subcore_barrier()

    @pl.when(s == 0)
    def _():
        pltpu.sync_copy(coeffs_sh, coeffs_out.at[c])


def _sc_atten(e3, ax, arc, zn):
    kern = functools.partial(
        pl.kernel,
        out_type=(jax.ShapeDtypeStruct((NW, SLAB * NSLAB, CH), jnp.float32),
                  jax.ShapeDtypeStruct((NC, N2), jnp.float32)),
        mesh=_sc_mesh(),
        compiler_params=pltpu.CompilerParams(needs_layout_passes=False),
        scratch_types=[
            pltpu.VMEM((3 * SLAB, CH), jnp.int32),
            pltpu.VMEM((SLAB, CH), jnp.float32),
            pltpu.VMEM((N,), jnp.float32),
            pltpu.VMEM((RPAD,), jnp.float32),
            pltpu.VMEM_SHARED((N2,), jnp.float32),
            pltpu.SemaphoreType.DMA,
        ],
    )(_atten_body)
    return kern(e3, ax, arc, zn)


def _coeffsum_body(c_ref, out_ref):
    out_ref[...] = c_ref[0, :][None, :] + c_ref[1, :][None, :]


def _coeffsum(coeffs2):
    out = pl.pallas_call(
        _coeffsum_body,
        out_shape=jax.ShapeDtypeStruct((1, N2), jnp.float32),
    )(coeffs2)
    return out[0]


def _agg_body(src_hbm, at2_hbm, atten_hbm, coeffs_hbm,
              xm_hbm, rm_hbm,
              acc_out,
              src_v, src_w, comb_v, atten_v, cs_v,
              xr0, xr1, zbuf, rm_sh, acc_sh, gx0, gx1, grm, scs, ssem):
    c = lax.axis_index("c")
    s = lax.axis_index("s")
    wid = c * NS + s
    pltpu.sync_copy(coeffs_hbm, cs_v)

    @pl.when(s == 0)
    def _():
        pltpu.sync_copy(rm_hbm, rm_sh)

    zv = jnp.zeros((L,), jnp.float32)

    def zb_body(i, _):
        for cc in range(D // L):
            zbuf[i, pl.ds(cc * L, L)] = zv
        return 0
    lax.fori_loop(0, ZR, zb_body, 0)

    def zi_body(j, _):
        pltpu.sync_copy(zbuf, acc_sh.at[pl.ds(s * RQ + j * ZR, ZR)])
        return 0
    lax.fori_loop(0, RQ // ZR, zi_body, 0)

    @pl.when(s == 0)
    def _():
        def zt_body(j, _):
            pltpu.sync_copy(zbuf, acc_sh.at[pl.ds(RQ * NS + j * ZR, ZR)])
            return 0
        lax.fori_loop(0, RTAIL // ZR, zt_body, 0)
    plsc.subcore_barrier()

    bufs = (xr0, xr1)
    gsems = (gx0, gx1)

    def _phase(i, p, me):
        bufp, bufo = bufs[p], bufs[1 - p]
        gsp, gso = gsems[p], gsems[1 - p]

        @pl.when(i > 0)
        def _():
            pltpu.make_async_copy(
                bufo, acc_sh.at[comb_v.at[SLAB + i - 1]], scs).wait()

        @pl.when(i + 1 < SLAB)
        def _():
            for q in range(4):
                pltpu.async_copy(
                    xm_hbm.at[me.at[i + 1, pl.ds(q * CH4, CH4)]],
                    bufo.at[pl.ds(q * CH4, CH4)], gso)

        for q in range(4):
            pltpu.make_async_copy(
                xm_hbm.at[me.at[i, pl.ds(q * CH4, CH4)]],
                bufp.at[pl.ds(q * CH4, CH4)], gsp).wait()
        rmcp = pltpu.async_copy(rm_sh.at[comb_v.at[i]], bufp, grm,
                                add=True)

        def w_body(g, _):
            sl = pl.ds(g * L, L)
            cg = plsc.load_gather(cs_v, [comb_v[SLAB + i, sl]])
            atten_v[i, sl] = atten_v[i, sl] / cg
            return 0
        lax.fori_loop(0, CH // L, w_body, 0)
        rmcp.wait()

        def e_body(g, _):
            w16 = atten_v[i, pl.ds(g * L, L)]
            for jj in range(L):
                e = g * L + jj
                w = w16[jj]
                for cc in range(D // L):
                    sl = pl.ds(cc * L, L)
                    bufp[e, sl] = bufp[e, sl] * w
            return 0
        lax.fori_loop(0, CH // L, e_body, 0)

        pltpu.async_copy(bufp, acc_sh.at[comb_v.at[SLAB + i]], scs,
                         add=True)

    def _half(j, me, other):
        sl8 = pl.ds(j * SLAB, SLAB)
        pltpu.sync_copy(at2_hbm.at[wid, j], comb_v)
        pltpu.sync_copy(atten_hbm.at[wid, sl8], atten_v)
        nsl8 = pl.ds((j + 1) * SLAB, SLAB)

        @pl.when(j + 1 < NSLAB)
        def _():
            pltpu.async_copy(src_hbm.at[wid, nsl8], other, ssem)

        def jj_body(jj, _):
            _phase(2 * jj, 0, me)
            _phase(2 * jj + 1, 1, me)
            return 0
        lax.fori_loop(0, SLAB // 2, jj_body, 0)

        pltpu.make_async_copy(
            xr1, acc_sh.at[comb_v.at[2 * SLAB - 1]], scs).wait()

        @pl.when(j + 1 < NSLAB)
        def _():
            pltpu.make_async_copy(src_hbm.at[wid, nsl8], other, ssem).wait()
            for q in range(4):
                pltpu.async_copy(
                    xm_hbm.at[other.at[0, pl.ds(q * CH4, CH4)]],
                    xr0.at[pl.ds(q * CH4, CH4)], gx0)

    pltpu.sync_copy(src_hbm.at[wid, pl.ds(0, SLAB)], src_v)
    for q in range(4):
        pltpu.async_copy(xm_hbm.at[src_v.at[0, pl.ds(q * CH4, CH4)]],
                         xr0.at[pl.ds(q * CH4, CH4)], gx0)

    def pair_body(j2, _):
        _half(2 * j2, src_v, src_w)
        _half(2 * j2 + 1, src_w, src_v)
        return 0
    lax.fori_loop(0, NSLAB // 2, pair_body, 0)

    plsc.subcore_barrier()
    pltpu.sync_copy(acc_sh.at[pl.ds(s * RQ, RQ)],
                    acc_out.at[c, pl.ds(s * RQ, RQ)])

    @pl.when(s == 0)
    def _():
        pltpu.sync_copy(acc_sh.at[pl.ds(RQ * NS, RTAIL)],
                        acc_out.at[c, pl.ds(RQ * NS, RTAIL)])


def _sc_aggregate(src, at2, atten, coeffs, xm, rm):
    kern = functools.partial(
        pl.kernel,
        out_type=jax.ShapeDtypeStruct((NC, N2, D), jnp.float32),
        mesh=_sc_mesh(),
        compiler_params=pltpu.CompilerParams(needs_layout_passes=False),
        scratch_types=[
            pltpu.VMEM((SLAB, CH), jnp.int32),
            pltpu.VMEM((SLAB, CH), jnp.int32),
            pltpu.VMEM((2 * SLAB, CH), jnp.int32),
            pltpu.VMEM((SLAB, CH), jnp.float32),
            pltpu.VMEM((N2,), jnp.float32),
            pltpu.VMEM((CH, D), jnp.float32),
            pltpu.VMEM((CH, D), jnp.float32),
            pltpu.VMEM((ZR, D), jnp.float32),
            pltpu.VMEM_SHARED((R, D), jnp.float32),
            pltpu.VMEM_SHARED((N2, D), jnp.float32),
            pltpu.SemaphoreType.DMA,
            pltpu.SemaphoreType.DMA,
            pltpu.SemaphoreType.DMA,
            pltpu.SemaphoreType.DMA,
            pltpu.SemaphoreType.DMA,
        ],
    )(_agg_body)
    return kern(src, at2, atten, coeffs, xm, rm)


def _epilogue_body(acc_ref, g_ref, b_ref, out_ref):
    sacc = acc_ref[0, pl.ds(0, N), :] + acc_ref[1, pl.ds(0, N), :]
    mu = jnp.mean(sacc, axis=0, keepdims=True)
    var = jnp.mean((sacc - mu) ** 2, axis=0, keepdims=True)
    out_ref[...] = jnp.tanh(
        (sacc - mu) / jnp.sqrt(var + EPS) * g_ref[...] + b_ref[...])


def _epilogue(acc2, eg, eb):
    return pl.pallas_call(
        _epilogue_body,
        out_shape=jax.ShapeDtypeStruct((N, D), jnp.float32),
    )(acc2, eg, eb)


def kernel(x, r, que_context, edge_index, edge_attr, edge_type,
           W_mess, b_mess, atten_weight, W_rel, b_rel,
           e_gamma, e_beta, r_gamma, r_beta):
    w1t = W_mess[:, :D].T
    w2t = W_mess[:, D:].T
    am = atten_weight[:, :D].T
    aq = atten_weight[:, D:]
    que = que_context[None, :]
    bm = b_mess[None, :]
    wrt = W_rel.T
    brl = b_rel[None, :]
    rg = r_gamma[None, :]
    rb = r_beta[None, :]
    eg = e_gamma[None, :]
    eb = e_beta[None, :]

    xm, ax2, rm, arc2, r_new = _prologue(
        x, w1t, w2t, am, aq, que, bm, r, wrt, brl, rg, rb)
    ax = ax2[:, 0]
    arc = jnp.concatenate([arc2[:, 0], jnp.zeros((RPAD - R,), jnp.float32)])

    pad = EP - E
    src = jnp.concatenate([edge_index[0], jnp.zeros((pad,), jnp.int32)])
    pad_tgt = N + (jnp.arange(pad, dtype=jnp.int32) % (N2 - N))
    tgt = jnp.concatenate([edge_index[1], pad_tgt])
    attr = jnp.concatenate([edge_attr, jnp.zeros((pad,), jnp.int32)])
    src4 = src.reshape(NW, NSLAB, SLAB, CH)
    tgt4 = tgt.reshape(NW, NSLAB, SLAB, CH)
    attr4 = attr.reshape(NW, NSLAB, SLAB, CH)
    e3 = jnp.concatenate([src4, attr4, tgt4], axis=2)
    at2 = jnp.concatenate([attr4, tgt4], axis=2)
    src = src4.reshape(NW, SLAB * NSLAB, CH)

    zn = jnp.zeros((N2,), jnp.float32)
    atten, coeffs2 = _sc_atten(e3, ax, arc, zn)
    coeffs = _coeffsum(coeffs2)

    acc2 = _sc_aggregate(src, at2, atten, coeffs, xm, rm)

    x_new = _epilogue(acc2, eg, eb)
    return (x_new, r_new)


# speedup vs baseline: 1.0579x; 1.0579x over previous
import functools

import jax
import jax.numpy as jnp
from jax import lax
from jax.experimental import pallas as pl
from jax.experimental.pallas import tpu as pltpu
from jax.experimental.pallas import tpu_sc as plsc

N = 10000
E = 320000
D = 128
R = 200
RPAD = 256
EPS = 1e-5

NC, NS, L = 2, 16, 16
NW = NC * NS
CH = 128
SLAB = 8
EPT = 10240
EP = NW * EPT
NSLAB = EPT // (SLAB * CH)
N2 = 10016
RQ = 624
RTAIL = N2 - RQ * NS
ZR = 8
CH2 = CH // 2
CH4 = CH // 4


def _tanh(v):
    return 1.0 - 2.0 / (jnp.exp(2.0 * v) + 1.0)


def _prologue_body(x_ref, w1t_ref, w2t_ref, am_ref, aq_ref, que_ref, bm_ref,
                   r_ref, wrt_ref, brl_ref, rg_ref, rb_ref,
                   xm_ref, ax_ref, rm_ref, arc_ref, rnew_ref):
    i = pl.program_id(0)
    xb = x_ref[...]
    xm = jnp.dot(xb, w1t_ref[...], preferred_element_type=jnp.float32)
    xm_ref[...] = xm
    ax_ref[...] = jnp.dot(xm, am_ref[...], preferred_element_type=jnp.float32)

    @pl.when(i == 0)
    def _():
        rb = r_ref[...]
        rm = jnp.dot(rb, w2t_ref[...], preferred_element_type=jnp.float32) \
            + bm_ref[...]
        rm_ref[...] = rm
        c0 = jnp.sum(que_ref[...] * aq_ref[...])
        arc_ref[...] = jnp.dot(rm, am_ref[...],
                               preferred_element_type=jnp.float32) + c0
        rl = jnp.dot(rb, wrt_ref[...], preferred_element_type=jnp.float32) \
            + brl_ref[...]
        mu = jnp.mean(rl, axis=0, keepdims=True)
        var = jnp.mean((rl - mu) ** 2, axis=0, keepdims=True)
        rnew_ref[...] = jnp.tanh(
            (rl - mu) / jnp.sqrt(var + EPS) * rg_ref[...] + rb_ref[...])


def _prologue(x, w1t, w2t, am, aq, que, bm, r, wrt, brl, rg, rb):
    grid = (N // 1000,)
    full = lambda shp: pl.BlockSpec(shp, lambda i: (0, 0))
    return pl.pallas_call(
        _prologue_body,
        grid=grid,
        in_specs=[
            pl.BlockSpec((1000, D), lambda i: (i, 0)),
            full((D, D)), full((D, D)), full((D, 1)), full((1, D)),
            full((1, D)), full((1, D)),
            full((R, D)), full((D, D)), full((1, D)), full((1, D)),
            full((1, D)),
        ],
        out_specs=[
            pl.BlockSpec((1000, D), lambda i: (i, 0)),
            pl.BlockSpec((1000, 1), lambda i: (i, 0)),
            full((R, D)), full((R, 1)), full((R, D)),
        ],
        out_shape=[
            jax.ShapeDtypeStruct((N, D), jnp.float32),
            jax.ShapeDtypeStruct((N, 1), jnp.float32),
            jax.ShapeDtypeStruct((R, D), jnp.float32),
            jax.ShapeDtypeStruct((R, 1), jnp.float32),
            jax.ShapeDtypeStruct((R, D), jnp.float32),
        ],
    )(x, w1t, w2t, am, aq, que, bm, r, wrt, brl, rg, rb)


def _sc_mesh():
    return plsc.VectorSubcoreMesh(core_axis_name="c", subcore_axis_name="s",
                                  num_cores=NC, num_subcores=NS)


def _atten_body(src_hbm, attr_hbm, tgt_hbm, ax_hbm, arc_hbm, zn_hbm,
                atten_out, coeffs_out,
                src_v, attr_v, tgt_v, atten_v, ax_v, arc_v, coeffs_sh, sem):
    c = lax.axis_index("c")
    s = lax.axis_index("s")
    wid = c * NS + s
    pltpu.sync_copy(ax_hbm, ax_v)
    pltpu.sync_copy(arc_hbm, arc_v)

    @pl.when(s == 0)
    def _():
        pltpu.sync_copy(zn_hbm, coeffs_sh)
    plsc.subcore_barrier()

    def slab_body(j, _):
        sl8 = pl.ds(j * SLAB, SLAB)
        pltpu.sync_copy(src_hbm.at[wid, sl8], src_v)
        pltpu.sync_copy(attr_hbm.at[wid, sl8], attr_v)
        pltpu.sync_copy(tgt_hbm.at[wid, sl8], tgt_v)

        def row_body(i, _):
            def vec_body(g, _):
                sl = pl.ds(g * L, L)
                axg = plsc.load_gather(ax_v, [src_v[i, sl]])
                arg = plsc.load_gather(arc_v, [attr_v[i, sl]])
                atten_v[i, sl] = jnp.exp(_tanh(axg + arg))
                return 0
            lax.fori_loop(0, CH // L, vec_body, 0)
            pltpu.async_copy(atten_v.at[i], coeffs_sh.at[tgt_v.at[i]],
                             sem, add=True)
            return 0
        lax.fori_loop(0, SLAB, row_body, 0)

        def drain_body(i, _):
            pltpu.make_async_copy(
                atten_v.at[i], coeffs_sh.at[tgt_v.at[i]], sem).wait()
            return 0
        lax.fori_loop(0, SLAB, drain_body, 0)
        pltpu.sync_copy(atten_v, atten_out.at[wid, sl8])
        return 0
    lax.fori_loop(0, NSLAB, slab_body, 0)

    plsc.subcore_barrier()

    @pl.when(s == 0)
    def _():
        pltpu.sync_copy(coeffs_sh, coeffs_out.at[c])


def _sc_atten(src, attr, tgt, ax, arc, zn):
    kern = functools.partial(
        pl.kernel,
        out_type=(jax.ShapeDtypeStruct((NW, SLAB * NSLAB, CH), jnp.float32),
                  jax.ShapeDtypeStruct((NC, N2), jnp.float32)),
        mesh=_sc_mesh(),
        compiler_params=pltpu.CompilerParams(needs_layout_passes=False),
        scratch_types=[
            pltpu.VMEM((SLAB, CH), jnp.int32),
            pltpu.VMEM((SLAB, CH), jnp.int32),
            pltpu.VMEM((SLAB, CH), jnp.int32),
            pltpu.VMEM((SLAB, CH), jnp.float32),
            pltpu.VMEM((N,), jnp.float32),
            pltpu.VMEM((RPAD,), jnp.float32),
            pltpu.VMEM_SHARED((N2,), jnp.float32),
            pltpu.SemaphoreType.DMA,
        ],
    )(_atten_body)
    return kern(src, attr, tgt, ax, arc, zn)


def _coeffsum_body(c_ref, out_ref):
    out_ref[...] = c_ref[0, :][None, :] + c_ref[1, :][None, :]


def _coeffsum(coeffs2):
    out = pl.pallas_call(
        _coeffsum_body,
        out_shape=jax.ShapeDtypeStruct((1, N2), jnp.float32),
    )(coeffs2)
    return out[0]


def _agg_body(src_hbm, attr_hbm, tgt_hbm, atten_hbm, coeffs_hbm,
              xm_hbm, rm_hbm,
              acc_out,
              src_v, src_w, attr_v, tgt_v, atten_v, cs_v,
              xr0, xr1, zbuf, rm_sh, acc_sh, gx0, gx1, grm, scs, ssem):
    c = lax.axis_index("c")
    s = lax.axis_index("s")
    wid = c * NS + s
    pltpu.sync_copy(coeffs_hbm, cs_v)

    @pl.when(s == 0)
    def _():
        pltpu.sync_copy(rm_hbm, rm_sh)

    zv = jnp.zeros((L,), jnp.float32)

    def zb_body(i, _):
        for cc in range(D // L):
            zbuf[i, pl.ds(cc * L, L)] = zv
        return 0
    lax.fori_loop(0, ZR, zb_body, 0)

    def zi_body(j, _):
        pltpu.sync_copy(zbuf, acc_sh.at[pl.ds(s * RQ + j * ZR, ZR)])
        return 0
    lax.fori_loop(0, RQ // ZR, zi_body, 0)

    @pl.when(s == 0)
    def _():
        def zt_body(j, _):
            pltpu.sync_copy(zbuf, acc_sh.at[pl.ds(RQ * NS + j * ZR, ZR)])
            return 0
        lax.fori_loop(0, RTAIL // ZR, zt_body, 0)
    plsc.subcore_barrier()

    bufs = (xr0, xr1)
    gsems = (gx0, gx1)

    def _phase(i, p, me):
        bufp, bufo = bufs[p], bufs[1 - p]
        gsp, gso = gsems[p], gsems[1 - p]

        @pl.when(i > 0)
        def _():
            pltpu.make_async_copy(
                bufo, acc_sh.at[tgt_v.at[i - 1]], scs).wait()

        @pl.when(i + 1 < SLAB)
        def _():
            for q in range(4):
                pltpu.async_copy(
                    xm_hbm.at[me.at[i + 1, pl.ds(q * CH4, CH4)]],
                    bufo.at[pl.ds(q * CH4, CH4)], gso)

        for q in range(4):
            pltpu.make_async_copy(
                xm_hbm.at[me.at[i, pl.ds(q * CH4, CH4)]],
                bufp.at[pl.ds(q * CH4, CH4)], gsp).wait()
        rmcp = pltpu.async_copy(rm_sh.at[attr_v.at[i]], bufp, grm,
                                add=True)

        def w_body(g, _):
            sl = pl.ds(g * L, L)
            cg = plsc.load_gather(cs_v, [tgt_v[i, sl]])
            atten_v[i, sl] = atten_v[i, sl] / cg
            return 0
        lax.fori_loop(0, CH // L, w_body, 0)
        rmcp.wait()

        def e_body(g, _):
            w16 = atten_v[i, pl.ds(g * L, L)]
            for jj in range(L):
                e = g * L + jj
                w = w16[jj]
                for cc in range(D // L):
                    sl = pl.ds(cc * L, L)
                    bufp[e, sl] = bufp[e, sl] * w
            return 0
        lax.fori_loop(0, CH // L, e_body, 0)

        pltpu.async_copy(bufp, acc_sh.at[tgt_v.at[i]], scs, add=True)

    def _half(j, me, other):
        sl8 = pl.ds(j * SLAB, SLAB)
        pltpu.sync_copy(attr_hbm.at[wid, sl8], attr_v)
        pltpu.sync_copy(tgt_hbm.at[wid, sl8], tgt_v)
        pltpu.sync_copy(atten_hbm.at[wid, sl8], atten_v)
        nsl8 = pl.ds((j + 1) * SLAB, SLAB)

        @pl.when(j + 1 < NSLAB)
        def _():
            pltpu.async_copy(src_hbm.at[wid, nsl8], other, ssem)

        def jj_body(jj, _):
            _phase(2 * jj, 0, me)
            _phase(2 * jj + 1, 1, me)
            return 0
        lax.fori_loop(0, SLAB // 2, jj_body, 0)

        pltpu.make_async_copy(
            xr1, acc_sh.at[tgt_v.at[SLAB - 1]], scs).wait()

        @pl.when(j + 1 < NSLAB)
        def _():
            pltpu.make_async_copy(src_hbm.at[wid, nsl8], other, ssem).wait()
            for q in range(4):
                pltpu.async_copy(
                    xm_hbm.at[other.at[0, pl.ds(q * CH4, CH4)]],
                    xr0.at[pl.ds(q * CH4, CH4)], gx0)

    pltpu.sync_copy(src_hbm.at[wid, pl.ds(0, SLAB)], src_v)
    for q in range(4):
        pltpu.async_copy(xm_hbm.at[src_v.at[0, pl.ds(q * CH4, CH4)]],
                         xr0.at[pl.ds(q * CH4, CH4)], gx0)

    def pair_body(j2, _):
        _half(2 * j2, src_v, src_w)
        _half(2 * j2 + 1, src_w, src_v)
        return 0
    lax.fori_loop(0, NSLAB // 2, pair_body, 0)

    plsc.subcore_barrier()
    pltpu.sync_copy(acc_sh.at[pl.ds(s * RQ, RQ)],
                    acc_out.at[c, pl.ds(s * RQ, RQ)])

    @pl.when(s == 0)
    def _():
        pltpu.sync_copy(acc_sh.at[pl.ds(RQ * NS, RTAIL)],
                        acc_out.at[c, pl.ds(RQ * NS, RTAIL)])


def _sc_aggregate(src, attr, tgt, atten, coeffs, xm, rm):
    kern = functools.partial(
        pl.kernel,
        out_type=jax.ShapeDtypeStruct((NC, N2, D), jnp.float32),
        mesh=_sc_mesh(),
        compiler_params=pltpu.CompilerParams(needs_layout_passes=False),
        scratch_types=[
            pltpu.VMEM((SLAB, CH), jnp.int32),
            pltpu.VMEM((SLAB, CH), jnp.int32),
            pltpu.VMEM((SLAB, CH), jnp.int32),
            pltpu.VMEM((SLAB, CH), jnp.int32),
            pltpu.VMEM((SLAB, CH), jnp.float32),
            pltpu.VMEM((N2,), jnp.float32),
            pltpu.VMEM((CH, D), jnp.float32),
            pltpu.VMEM((CH, D), jnp.float32),
            pltpu.VMEM((ZR, D), jnp.float32),
            pltpu.VMEM_SHARED((R, D), jnp.float32),
            pltpu.VMEM_SHARED((N2, D), jnp.float32),
            pltpu.SemaphoreType.DMA,
            pltpu.SemaphoreType.DMA,
            pltpu.SemaphoreType.DMA,
            pltpu.SemaphoreType.DMA,
            pltpu.SemaphoreType.DMA,
        ],
    )(_agg_body)
    return kern(src, attr, tgt, atten, coeffs, xm, rm)


def _epilogue_body(acc_ref, g_ref, b_ref, out_ref):
    sacc = acc_ref[0, pl.ds(0, N), :] + acc_ref[1, pl.ds(0, N), :]
    mu = jnp.mean(sacc, axis=0, keepdims=True)
    var = jnp.mean((sacc - mu) ** 2, axis=0, keepdims=True)
    out_ref[...] = jnp.tanh(
        (sacc - mu) / jnp.sqrt(var + EPS) * g_ref[...] + b_ref[...])


def _epilogue(acc2, eg, eb):
    return pl.pallas_call(
        _epilogue_body,
        out_shape=jax.ShapeDtypeStruct((N, D), jnp.float32),
    )(acc2, eg, eb)


def kernel(x, r, que_context, edge_index, edge_attr, edge_type,
           W_mess, b_mess, atten_weight, W_rel, b_rel,
           e_gamma, e_beta, r_gamma, r_beta):
    w1t = W_mess[:, :D].T
    w2t = W_mess[:, D:].T
    am = atten_weight[:, :D].T
    aq = atten_weight[:, D:]
    que = que_context[None, :]
    bm = b_mess[None, :]
    wrt = W_rel.T
    brl = b_rel[None, :]
    rg = r_gamma[None, :]
    rb = r_beta[None, :]
    eg = e_gamma[None, :]
    eb = e_beta[None, :]

    xm, ax2, rm, arc2, r_new = _prologue(
        x, w1t, w2t, am, aq, que, bm, r, wrt, brl, rg, rb)
    ax = ax2[:, 0]
    arc = jnp.concatenate([arc2[:, 0], jnp.zeros((RPAD - R,), jnp.float32)])

    pad = EP - E
    src = jnp.concatenate([edge_index[0], jnp.zeros((pad,), jnp.int32)])
    pad_tgt = N + (jnp.arange(pad, dtype=jnp.int32) % (N2 - N))
    tgt = jnp.concatenate([edge_index[1], pad_tgt])
    attr = jnp.concatenate([edge_attr, jnp.zeros((pad,), jnp.int32)])
    src = src.reshape(NW, SLAB * NSLAB, CH)
    tgt = tgt.reshape(NW, SLAB * NSLAB, CH)
    attr = attr.reshape(NW, SLAB * NSLAB, CH)

    zn = jnp.zeros((N2,), jnp.float32)
    atten, coeffs2 = _sc_atten(src, attr, tgt, ax, arc, zn)
    coeffs = _coeffsum(coeffs2)

    acc2 = _sc_aggregate(src, attr, tgt, atten, coeffs, xm, rm)

    x_new = _epilogue(acc2, eg, eb)
    return (x_new, r_new)
